# Initial kernel scaffold; baseline (speedup 1.0000x reference)
import jax
import jax.numpy as jnp
from jax.experimental import pallas as pl


def kernel(a, b, e, W1, b1, W2, b2):
    raise NotImplementedError("write your pallas kernel here")


# speedup vs baseline: 34.4084x; 34.4084x over previous
import functools

import jax
import jax.numpy as jnp
from jax import lax
from jax.experimental import pallas as pl
from jax.experimental.pallas import tpu as pltpu
from jax.experimental.pallas import tpu_sc as plsc

B, N, DEG, F = 8, 2048, 16, 128
NC, NS, L = 2, 16, 16
NW = NC * NS
FPW = 8
NGF = F // FPW
NGN = NW // NGF
NPW = N // NGN
RG = NPW // L


def _make_sc_stage(is_sum: bool):
    mesh = plsc.VectorSubcoreMesh(core_axis_name="c", subcore_axis_name="s")

    @functools.partial(
        pl.kernel,
        mesh=mesh,
        compiler_params=pltpu.CompilerParams(needs_layout_passes=False),
        out_type=jax.ShapeDtypeStruct((B, F, N), jnp.float32),
        scratch_types=[
            pltpu.VMEM((FPW * N,), jnp.float32),
            pltpu.VMEM((DEG, NPW), jnp.int32),
            pltpu.VMEM((FPW, NPW), jnp.float32),
        ],
    )
    def stage(z_hbm, e_hbm, out_hbm, table_v, e_v, out_v):
        wid = lax.axis_index("s") * NC + lax.axis_index("c")
        f0 = (wid // NGN) * FPW
        n0 = (wid % NGN) * NPW

        def batch_body(b, _):
            pltpu.sync_copy(z_hbm.at[b, pl.ds(f0 * N, FPW * N)], table_v)
            pltpu.sync_copy(e_hbm.at[b, :, pl.ds(n0, NPW)], e_v)

            def rg_body(ri, _):
                r = ri * L
                accs = [table_v[pl.ds(f * N + n0 + r, L)] for f in range(FPW)]
                for d in range(DEG):
                    ev = e_v[d, pl.ds(r, L)]
                    for f in range(FPW):
                        g = plsc.load_gather(table_v, [ev + (f * N)])
                        if is_sum:
                            accs[f] = accs[f] + g
                        else:
                            accs[f] = jnp.maximum(accs[f], g)
                for f in range(FPW):
                    v = accs[f]
                    if is_sum:
                        v = jnp.maximum(v, 0.0)
                    out_v[f, pl.ds(r, L)] = v
                return 0

            lax.fori_loop(0, RG, rg_body, 0)
            pltpu.sync_copy(out_v,
                            out_hbm.at[b, pl.ds(f0, FPW), pl.ds(n0, NPW)])
            return 0

        lax.fori_loop(0, B, batch_body, 0)

    return stage


_sc_sum_relu = _make_sc_stage(True)
_sc_max = _make_sc_stage(False)


def _mm_kernel_nf(x_ref, w_ref, bias_ref, o_ref):
    z = lax.dot_general(w_ref[...], x_ref[0],
                        (((0,), (1,)), ((), ())),
                        preferred_element_type=jnp.float32)
    o_ref[0] = z + bias_ref[...]


def _mm_kernel_fn(x_ref, w_ref, bias_ref, o_ref):
    z = lax.dot_general(w_ref[...], x_ref[0],
                        (((0,), (0,)), ((), ())),
                        preferred_element_type=jnp.float32)
    o_ref[0] = z + bias_ref[...]


def _mm_call(body, x, w, bias, d1, d2):
    return pl.pallas_call(
        body,
        grid=(B,),
        in_specs=[
            pl.BlockSpec((1, d1, d2), lambda b: (b, 0, 0)),
            pl.BlockSpec((F, F), lambda b: (0, 0)),
            pl.BlockSpec((F, 1), lambda b: (0, 0)),
        ],
        out_specs=pl.BlockSpec((1, F, N), lambda b: (b, 0, 0)),
        out_shape=jax.ShapeDtypeStruct((B, F, N), jnp.float32),
    )(x, w, bias)


def _sum_kernel(x_ref, o_ref):
    o_ref[...] = jnp.sum(x_ref[...], axis=-1)


def _sum_nodes(p):
    return pl.pallas_call(
        _sum_kernel,
        out_shape=jax.ShapeDtypeStruct((B, F), jnp.float32),
    )(p)


def kernel(a, b, e, W1, b1, W2, b2):
    del b
    e_t = jnp.transpose(e, (0, 2, 1)).astype(jnp.int32)
    bias1 = (b1 / 17.0).reshape(F, 1).astype(jnp.float32)
    bias2 = (b2 / 17.0).reshape(F, 1).astype(jnp.float32)

    z1 = _mm_call(_mm_kernel_nf, a, W1, bias1, N, F)
    h1 = _sc_sum_relu(z1.reshape(B, F * N), e_t)
    p1 = _sc_max(h1.reshape(B, F * N), e_t)
    z2 = _mm_call(_mm_kernel_fn, p1, W2, bias2, F, N)
    h2 = _sc_sum_relu(z2.reshape(B, F * N), e_t)
    p2 = _sc_max(h2.reshape(B, F * N), e_t)
    return _sum_nodes(p2)
